# baseline (speedup 1.0000x reference)
import functools

import jax
import jax.numpy as jnp
from jax import lax
from jax.experimental import pallas as pl
from jax.experimental.pallas import tpu as pltpu
from jax.experimental.pallas import tpu_sc as plsc

_B = 4096
_S = 200
_H = 64
_NW = 32
_BPW = _B // _NW
_STREAMS = ((0, 104), (104, 96))
_GRP = 8
_EPS = 1e-12
_MAGIC = 0x5F3759DF

_GATHER_DNUMS = lax.GatherDimensionNumbers(
    offset_dims=(), collapsed_slice_dims=(0,), start_index_map=(0,))


def _allsum16(v, perms):
    for p in perms:
        v = v + lax.gather(v, p, _GATHER_DNUMS, (1,),
                           mode=lax.GatherScatterMode.PROMISE_IN_BOUNDS)
    return v


def _rsqrt16(a):
    ai = lax.bitcast_convert_type(a, jnp.int32)
    yi = jnp.int32(_MAGIC) - (ai >> 1)
    y = lax.bitcast_convert_type(yi, jnp.float32)
    ha = a * jnp.float32(0.5)
    for _ in range(3):
        y = y * (jnp.float32(1.5) - ha * y * y)
    return y


def _body(ids_hbm, wemb_hbm, pos_hbm, gam_hbm, bet_hbm, out_hbm,
          idx_v, rows0, rows1, out0, out1, pos_v, g_v, b_v,
          sem_g, sem_o):
    wid = lax.axis_index("s") * 2 + lax.axis_index("c")
    b0 = pl.multiple_of(wid * _BPW, _BPW)

    pltpu.sync_copy(pos_hbm, pos_v)
    pltpu.sync_copy(gam_hbm, g_v)
    pltpu.sync_copy(bet_hbm, b_v)
    pltpu.sync_copy(ids_hbm.at[pl.ds(b0, _GRP), :], idx_v)
    gk = [g_v[pl.ds(k * 16, 16)] for k in range(4)]
    bk = [b_v[pl.ds(k * 16, 16)] for k in range(4)]

    inv_h = jnp.float32(1.0 / _H)
    perms = [(lax.iota(jnp.int32, 16) ^ jnp.int32(k))[:, None]
             for k in (1, 2, 4, 8)]
    rows_bufs = (rows0, rows1)
    out_bufs = (out0, out1)
    drain_rows_src = wemb_hbm.at[pl.ds(0, _S)]
    drain_out_src = out_hbm.at[pl.ds(0, _S * _H)]

    def fire_gather(c, buf):
        row = c % _GRP
        for off, n in _STREAMS:
            pltpu.async_copy(
                wemb_hbm.at[idx_v.at[row, pl.ds(off, n)]],
                buf.at[pl.ds(off, n)], sem_g)

    fire_gather(0, rows0)

    def seq_pair(p, carry):
        for b in range(2):
            rows_b = rows_bufs[b]
            out_b = out_bufs[b]
            c = p * 2 + b
            nxt = c + 1

            pltpu.make_async_copy(drain_rows_src, rows_b, sem_g).wait()

            @pl.when(jnp.logical_and(nxt % _GRP == 0, nxt < _BPW))
            def _():
                pltpu.sync_copy(ids_hbm.at[pl.ds(b0 + nxt, _GRP), :], idx_v)

            @pl.when(nxt < _BPW)
            def _():
                fire_gather(nxt, rows_bufs[1 - b])

            @pl.when(p > 0)
            def _():
                pltpu.make_async_copy(drain_out_src, out_b, sem_o).wait()

            @plsc.parallel_loop(0, _S, unroll=4)
            def row_body(t):
                x = [rows_b[t, pl.ds(k * 16, 16)]
                     + pos_v[t, pl.ds(k * 16, 16)] for k in range(4)]
                sv = (x[0] + x[1]) + (x[2] + x[3])
                q = (x[0] * x[0] + x[1] * x[1]) + (
                    x[2] * x[2] + x[3] * x[3])
                meanv = _allsum16(sv, perms) * inv_h
                varv = _allsum16(q, perms) * inv_h - meanv * meanv
                rstd = _rsqrt16(varv + jnp.float32(_EPS))
                for k in range(4):
                    y = (x[k] - meanv) * rstd * gk[k] + bk[k]
                    out_b[pl.ds(t * _H + k * 16, 16)] = y

            pltpu.async_copy(
                out_b, out_hbm.at[pl.ds((b0 + c) * _S * _H, _S * _H)],
                sem_o)
        return carry

    lax.fori_loop(0, _BPW // 2, seq_pair, 0, unroll=False)

    pltpu.make_async_copy(drain_out_src, out0, sem_o).wait()
    pltpu.make_async_copy(drain_out_src, out1, sem_o).wait()


_emb_ln = functools.partial(
    pl.kernel,
    mesh=plsc.VectorSubcoreMesh(core_axis_name="c", subcore_axis_name="s"),
    compiler_params=pltpu.CompilerParams(use_tc_tiling_on_sc=False),
    out_type=jax.ShapeDtypeStruct((_B * _S * _H,), jnp.float32),
    scratch_types=[
        pltpu.VMEM((_GRP, _S), jnp.int32),
        pltpu.VMEM((_S, _H), jnp.float32),
        pltpu.VMEM((_S, _H), jnp.float32),
        pltpu.VMEM((_S * _H,), jnp.float32),
        pltpu.VMEM((_S * _H,), jnp.float32),
        pltpu.VMEM((_S, _H), jnp.float32),
        pltpu.VMEM((_H,), jnp.float32),
        pltpu.VMEM((_H,), jnp.float32),
        pltpu.SemaphoreType.DMA,
        pltpu.SemaphoreType.DMA,
    ],
)(_body)


def kernel(input_ids, word_emb, pos_emb, ln_gamma, ln_beta):
    if input_ids.dtype != jnp.int32:
        input_ids = input_ids.astype(jnp.int32)
    flat = _emb_ln(input_ids, word_emb, pos_emb, ln_gamma, ln_beta)
    return flat.reshape(_B, _S, _H)


# speedup vs baseline: 1.2267x; 1.2267x over previous
import functools

import jax
import jax.numpy as jnp
from jax import lax
from jax.experimental import pallas as pl
from jax.experimental.pallas import tpu as pltpu
from jax.experimental.pallas import tpu_sc as plsc

_B = 4096
_S = 200
_H = 64
_NW = 32
_BPW = _B // _NW
_STREAMS = ((0, 128), (128, 72))
_HP = 128
_GRP = 8
_EPS = 1e-12
_MAGIC = 0x5F3759DF

_GATHER_DNUMS = lax.GatherDimensionNumbers(
    offset_dims=(), collapsed_slice_dims=(0,), start_index_map=(0,))


def _allsum16(v, perms):
    for p in perms:
        v = v + lax.gather(v, p, _GATHER_DNUMS, (1,),
                           mode=lax.GatherScatterMode.PROMISE_IN_BOUNDS)
    return v


def _rsqrt16(a):
    ai = lax.bitcast_convert_type(a, jnp.int32)
    yi = jnp.int32(_MAGIC) - (ai >> 1)
    y = lax.bitcast_convert_type(yi, jnp.float32)
    ha = a * jnp.float32(0.5)
    for _ in range(3):
        y = y * (jnp.float32(1.5) - ha * y * y)
    return y


def _body(ids_hbm, wemb_hbm, pos_hbm, gam_hbm, bet_hbm, out_hbm,
          idx_v, rows0, rows1, out0, out1, pos_v, g_v, b_v,
          sem_g, sem_o):
    wid = lax.axis_index("s") * 2 + lax.axis_index("c")
    b0 = pl.multiple_of(wid * _BPW, _BPW)

    pltpu.sync_copy(pos_hbm, pos_v)
    pltpu.sync_copy(gam_hbm, g_v)
    pltpu.sync_copy(bet_hbm, b_v)
    pltpu.sync_copy(ids_hbm.at[pl.ds(b0, _GRP), :], idx_v)
    gk = [g_v[pl.ds(k * 16, 16)] for k in range(4)]
    bk = [b_v[pl.ds(k * 16, 16)] for k in range(4)]

    inv_h = jnp.float32(1.0 / _H)
    perms = [(lax.iota(jnp.int32, 16) ^ jnp.int32(k))[:, None]
             for k in (1, 2, 4, 8)]
    rows_bufs = (rows0, rows1)
    out_bufs = (out0, out1)
    drain_rows_src = wemb_hbm.at[pl.ds(0, _S)]
    drain_out_src = out_hbm.at[0]

    def fire_gather(c, buf):
        row = c % _GRP
        for off, n in _STREAMS:
            pltpu.async_copy(
                wemb_hbm.at[idx_v.at[row, pl.ds(off, n)]],
                buf.at[pl.ds(off, n)], sem_g)

    fire_gather(0, rows0)

    def seq_pair(p, carry):
        for b in range(2):
            rows_b = rows_bufs[b]
            out_b = out_bufs[b]
            c = p * 2 + b
            nxt = c + 1

            pltpu.make_async_copy(drain_rows_src, rows_b, sem_g).wait()

            @pl.when(jnp.logical_and(nxt % _GRP == 0, nxt < _BPW))
            def _():
                off = pl.multiple_of(b0 + nxt, _GRP)
                pltpu.sync_copy(ids_hbm.at[pl.ds(off, _GRP), :], idx_v)

            @pl.when(nxt < _BPW)
            def _():
                fire_gather(nxt, rows_bufs[1 - b])

            @pl.when(p > 0)
            def _():
                pltpu.make_async_copy(drain_out_src, out_b, sem_o).wait()

            @plsc.parallel_loop(0, _S, unroll=4)
            def row_body(t):
                x = [rows_b[t, pl.ds(k * 16, 16)]
                     + pos_v[t, pl.ds(k * 16, 16)] for k in range(4)]
                sv = (x[0] + x[1]) + (x[2] + x[3])
                q = (x[0] * x[0] + x[1] * x[1]) + (
                    x[2] * x[2] + x[3] * x[3])
                meanv = _allsum16(sv, perms) * inv_h
                varv = _allsum16(q, perms) * inv_h - meanv * meanv
                rstd = _rsqrt16(varv + jnp.float32(_EPS))
                for k in range(4):
                    y = (x[k] - meanv) * rstd * gk[k] + bk[k]
                    out_b[t, pl.ds(k * 16, 16)] = y

            pltpu.async_copy(out_b, out_hbm.at[b0 + c], sem_o)
        return carry

    lax.fori_loop(0, _BPW // 2, seq_pair, 0, unroll=False)

    pltpu.make_async_copy(drain_out_src, out0, sem_o).wait()
    pltpu.make_async_copy(drain_out_src, out1, sem_o).wait()


_emb_ln = functools.partial(
    pl.kernel,
    mesh=plsc.VectorSubcoreMesh(core_axis_name="c", subcore_axis_name="s"),
    compiler_params=pltpu.CompilerParams(use_tc_tiling_on_sc=True),
    out_type=jax.ShapeDtypeStruct((_B, _S, _H), jnp.float32),
    scratch_types=[
        pltpu.VMEM((_GRP, _S), jnp.int32),
        pltpu.VMEM((_S, _HP), jnp.float32),
        pltpu.VMEM((_S, _HP), jnp.float32),
        pltpu.VMEM((_S, _H), jnp.float32),
        pltpu.VMEM((_S, _H), jnp.float32),
        pltpu.VMEM((_S, _H), jnp.float32),
        pltpu.VMEM((_H,), jnp.float32),
        pltpu.VMEM((_H,), jnp.float32),
        pltpu.SemaphoreType.DMA,
        pltpu.SemaphoreType.DMA,
    ],
)(_body)


def kernel(input_ids, word_emb, pos_emb, ln_gamma, ln_beta):
    if input_ids.dtype != jnp.int32:
        input_ids = input_ids.astype(jnp.int32)
    w128 = jnp.pad(word_emb, ((0, 0), (0, _HP - _H)))
    return _emb_ln(input_ids, w128, pos_emb, ln_gamma, ln_beta)


# speedup vs baseline: 1.2384x; 1.0095x over previous
import functools

import jax
import jax.numpy as jnp
from jax import lax
from jax.experimental import pallas as pl
from jax.experimental.pallas import tpu as pltpu
from jax.experimental.pallas import tpu_sc as plsc

_B = 4096
_S = 200
_H = 64
_NW = 32
_BPW = _B // _NW
_STREAMS = ((0, 128), (128, 72))
_HP = 128
_GRP = 8
_EPS = 1e-12
_MAGIC = 0x5F3759DF

_GATHER_DNUMS = lax.GatherDimensionNumbers(
    offset_dims=(), collapsed_slice_dims=(0,), start_index_map=(0,))


def _allsum16(v, perms):
    for p in perms:
        v = v + lax.gather(v, p, _GATHER_DNUMS, (1,),
                           mode=lax.GatherScatterMode.PROMISE_IN_BOUNDS)
    return v


def _rsqrt16(a):
    ai = lax.bitcast_convert_type(a, jnp.int32)
    yi = jnp.int32(_MAGIC) - (ai >> 1)
    y = lax.bitcast_convert_type(yi, jnp.float32)
    ha = a * jnp.float32(0.5)
    for _ in range(2):
        y = y * (jnp.float32(1.5) - ha * y * y)
    return y


def _body(ids_hbm, wemb_hbm, pos_hbm, gam_hbm, bet_hbm, out_hbm,
          idx_v, rows0, rows1, out0, out1, pos_v, g_v, b_v,
          sem_g, sem_o):
    wid = lax.axis_index("s") * 2 + lax.axis_index("c")
    b0 = pl.multiple_of(wid * _BPW, _BPW)

    pltpu.sync_copy(pos_hbm, pos_v)
    pltpu.sync_copy(gam_hbm, g_v)
    pltpu.sync_copy(bet_hbm, b_v)
    pltpu.sync_copy(ids_hbm.at[pl.ds(b0, _GRP), :], idx_v)
    gk = [g_v[pl.ds(k * 16, 16)] for k in range(4)]
    bk = [b_v[pl.ds(k * 16, 16)] for k in range(4)]

    inv_h = jnp.float32(1.0 / _H)
    perms = [(lax.iota(jnp.int32, 16) ^ jnp.int32(k))[:, None]
             for k in (1, 2, 4, 8)]
    rows_bufs = (rows0, rows1)
    out_bufs = (out0, out1)
    drain_rows_src = wemb_hbm.at[pl.ds(0, _S)]
    drain_out_src = out_hbm.at[0]

    def fire_gather(c, buf):
        row = c % _GRP
        for off, n in _STREAMS:
            pltpu.async_copy(
                wemb_hbm.at[idx_v.at[row, pl.ds(off, n)]],
                buf.at[pl.ds(off, n)], sem_g)

    fire_gather(0, rows0)

    def seq_pair(p, carry):
        for b in range(2):
            rows_b = rows_bufs[b]
            out_b = out_bufs[b]
            c = p * 2 + b
            nxt = c + 1

            pltpu.make_async_copy(drain_rows_src, rows_b, sem_g).wait()

            @pl.when(jnp.logical_and(nxt % _GRP == 0, nxt < _BPW))
            def _():
                off = pl.multiple_of(b0 + nxt, _GRP)
                pltpu.sync_copy(ids_hbm.at[pl.ds(off, _GRP), :], idx_v)

            @pl.when(nxt < _BPW)
            def _():
                fire_gather(nxt, rows_bufs[1 - b])

            @pl.when(p > 0)
            def _():
                pltpu.make_async_copy(drain_out_src, out_b, sem_o).wait()

            @plsc.parallel_loop(0, _S, unroll=4)
            def row_body(t):
                x = [rows_b[t, pl.ds(k * 16, 16)]
                     + pos_v[t, pl.ds(k * 16, 16)] for k in range(4)]
                sv = (x[0] + x[1]) + (x[2] + x[3])
                q = (x[0] * x[0] + x[1] * x[1]) + (
                    x[2] * x[2] + x[3] * x[3])
                meanv = _allsum16(sv, perms) * inv_h
                varv = _allsum16(q, perms) * inv_h - meanv * meanv
                rstd = _rsqrt16(varv + jnp.float32(_EPS))
                for k in range(4):
                    y = (x[k] - meanv) * rstd * gk[k] + bk[k]
                    out_b[t, pl.ds(k * 16, 16)] = y

            pltpu.async_copy(out_b, out_hbm.at[b0 + c], sem_o)
        return carry

    lax.fori_loop(0, _BPW // 2, seq_pair, 0, unroll=False)

    pltpu.make_async_copy(drain_out_src, out0, sem_o).wait()
    pltpu.make_async_copy(drain_out_src, out1, sem_o).wait()


_emb_ln = functools.partial(
    pl.kernel,
    mesh=plsc.VectorSubcoreMesh(core_axis_name="c", subcore_axis_name="s"),
    compiler_params=pltpu.CompilerParams(use_tc_tiling_on_sc=True),
    out_type=jax.ShapeDtypeStruct((_B, _S, _H), jnp.float32),
    scratch_types=[
        pltpu.VMEM((_GRP, _S), jnp.int32),
        pltpu.VMEM((_S, _HP), jnp.float32),
        pltpu.VMEM((_S, _HP), jnp.float32),
        pltpu.VMEM((_S, _H), jnp.float32),
        pltpu.VMEM((_S, _H), jnp.float32),
        pltpu.VMEM((_S, _H), jnp.float32),
        pltpu.VMEM((_H,), jnp.float32),
        pltpu.VMEM((_H,), jnp.float32),
        pltpu.SemaphoreType.DMA,
        pltpu.SemaphoreType.DMA,
    ],
)(_body)


def kernel(input_ids, word_emb, pos_emb, ln_gamma, ln_beta):
    if input_ids.dtype != jnp.int32:
        input_ids = input_ids.astype(jnp.int32)
    w128 = jnp.pad(word_emb, ((0, 0), (0, _HP - _H)))
    return _emb_ln(input_ids, w128, pos_emb, ln_gamma, ln_beta)
